# baseline (speedup 1.0000x reference)
import jax
import jax.numpy as jnp
from jax import lax
from jax.experimental import pallas as pl
from jax.experimental.pallas import tpu as pltpu
from jax.experimental.pallas import tpu_sc as plsc

_B = 4096
_L = 200
_H = 64
_NOUT = 100
_NC = 2
_NS = 16
_NW = _NC * _NS
_BPW = _B // _NW
_C1 = 128
_C2 = _L - _C1


def _splat(vec, u):
    return lax.gather(
        vec, jnp.full((16, 1), u, jnp.int32),
        lax.GatherDimensionNumbers(offset_dims=(), collapsed_slice_dims=(0,),
                                   start_index_map=(0,)),
        slice_sizes=(1,), mode=lax.GatherScatterMode.PROMISE_IN_BOUNDS)


def _embbag_body(text_hbm, table_hbm, out_hbm, idx_v, w_v, buf0, buf1, out_v,
                 sem0, sem1):
    c = lax.axis_index("c")
    s = lax.axis_index("s")
    wid = s * _NC + c
    base = wid * _BPW
    pltpu.sync_copy(text_hbm.at[pl.ds(base * _L, _BPW * _L)], idx_v)

    @pl.loop(0, _BPW * _L // 16)
    def _prep(j):
        off = pl.multiple_of(j * 16, 8)
        v = idx_v[pl.ds(off, 16)]
        idx_v[pl.ds(off, 16)] = lax.shift_right_logical(v, 1)
        w_v[pl.ds(off, 16)] = lax.convert_element_type(
            lax.bitwise_and(v, 1), jnp.float32)

    bufs = (buf0, buf1)
    sems = (sem0, sem1)

    def issue(i, slot):
        off = pl.multiple_of(i * _L, 8)
        pltpu.async_copy(table_hbm.at[idx_v.at[pl.ds(off, _C1)]],
                         bufs[slot].at[pl.ds(0, _C1)], sems[slot])
        pltpu.async_copy(table_hbm.at[idx_v.at[pl.ds(off + _C1, _C2)]],
                         bufs[slot].at[pl.ds(_C1, _C2)], sems[slot])

    def drain(slot):
        pltpu.make_async_copy(table_hbm.at[idx_v.at[pl.ds(0, _C1)]],
                              bufs[slot].at[pl.ds(0, _C1)], sems[slot]).wait()
        pltpu.make_async_copy(table_hbm.at[idx_v.at[pl.ds(0, _C2)]],
                              bufs[slot].at[pl.ds(_C1, _C2)], sems[slot]).wait()

    issue(0, 0)

    @pl.loop(0, _BPW, step=2)
    def _bag_pair(i):
        for b in range(2):
            ib = i + b
            buf = bufs[b]

            @pl.when(ib + 1 < _BPW)
            def _():
                issue(ib + 1, 1 - b)

            drain(b)

            woff = pl.multiple_of(ib * _L, 8)

            def red(r, acc):
                a = list(acc)
                wvec = w_v[pl.ds(woff + r * 16, 16)]
                for u in range(16):
                    j = r * 16 + u
                    wj = _splat(wvec, u)
                    for k in range(4):
                        lo = buf[j, pl.ds(k * 16, 16)]
                        hi = buf[j, pl.ds(_H + k * 16, 16)]
                        a[k] = a[k] + (lo + wj * (hi - lo))
                return tuple(a)

            acc = tuple(jnp.zeros((16,), jnp.float32) for _ in range(4))
            acc = lax.fori_loop(0, _L // 16, red, acc)

            def red_tail(r, acc):
                a = list(acc)
                wvec = w_v[pl.ds(woff + 192, 16)]
                for u in range(8):
                    j = 192 + u
                    wj = _splat(wvec, u)
                    for k in range(4):
                        lo = buf[j, pl.ds(k * 16, 16)]
                        hi = buf[j, pl.ds(_H + k * 16, 16)]
                        a[k] = a[k] + (lo + wj * (hi - lo))
                return tuple(a)

            acc = lax.fori_loop(0, 1, red_tail, acc)

            row = ib // 2
            colbase = (ib % 2) * _H
            for k in range(4):
                out_v[row, pl.ds(colbase + k * 16, 16)] = acc[k]

    pltpu.sync_copy(out_v, out_hbm.at[pl.ds(wid * (_BPW // 2), _BPW // 2)])


def _embbag_sums(text_flat, table_pairs):
    mesh = plsc.VectorSubcoreMesh(core_axis_name="c", subcore_axis_name="s",
                                  num_cores=_NC, num_subcores=_NS)
    f = pl.kernel(
        _embbag_body,
        out_type=jax.ShapeDtypeStruct((_B // 2, 2 * _H), jnp.float32),
        mesh=mesh,
        name="embbag_sums",
        scratch_types=[
            pltpu.VMEM((_BPW * _L,), jnp.int32),
            pltpu.VMEM((_BPW * _L,), jnp.float32),
            pltpu.VMEM((_L, 2 * _H), jnp.float32),
            pltpu.VMEM((_L, 2 * _H), jnp.float32),
            pltpu.VMEM((_BPW // 2, 2 * _H), jnp.float32),
            pltpu.SemaphoreType.DMA,
            pltpu.SemaphoreType.DMA,
        ],
        compiler_params=pltpu.CompilerParams(use_tc_tiling_on_sc=True),
    )
    return f(text_flat, table_pairs).reshape(_B, _H)


def _linear_body(x_ref, w_ref, b_ref, o_ref):
    o_ref[...] = (
        lax.dot_general(x_ref[...], w_ref[...], (((1,), (1,)), ((), ())),
                        preferred_element_type=jnp.float32) * (1.0 / _L)
        + b_ref[...]
    )


def kernel(text, emb_weight, fc1_weight, fc1_bias):
    table_pairs = emb_weight.reshape(500000, 2 * _H)
    sums = _embbag_sums(text.reshape(-1), table_pairs)
    out = pl.pallas_call(
        _linear_body,
        out_shape=jax.ShapeDtypeStruct((_B, _NOUT), jnp.float32),
    )(sums, fc1_weight, fc1_bias.reshape(1, _NOUT))
    return out


# speedup vs baseline: 1.8906x; 1.8906x over previous
import jax
import jax.numpy as jnp
from jax import lax
from jax.experimental import pallas as pl
from jax.experimental.pallas import tpu as pltpu
from jax.experimental.pallas import tpu_sc as plsc

_B = 4096
_L = 200
_H = 64
_NOUT = 100
_NC = 2
_NS = 16
_NW = _NC * _NS
_BPW = _B // _NW
_C1 = 128
_C2 = _L - _C1


def _embbag_body(text_hbm, table_hbm, out_hbm, idx_v, buf0, buf1, out_v,
                 sem0, sem1):
    c = lax.axis_index("c")
    s = lax.axis_index("s")
    wid = s * _NC + c
    base = wid * _BPW
    pltpu.sync_copy(text_hbm.at[pl.ds(base * _L, _BPW * _L)], idx_v)

    bufs = (buf0, buf1)
    sems = (sem0, sem1)

    def issue(i, slot):
        off = pl.multiple_of(i * _L, 8)
        pltpu.async_copy(table_hbm.at[idx_v.at[pl.ds(off, _C1)]],
                         bufs[slot].at[pl.ds(0, _C1)], sems[slot])
        pltpu.async_copy(table_hbm.at[idx_v.at[pl.ds(off + _C1, _C2)]],
                         bufs[slot].at[pl.ds(_C1, _C2)], sems[slot])

    def drain(slot):
        pltpu.make_async_copy(table_hbm.at[idx_v.at[pl.ds(0, _C1)]],
                              bufs[slot].at[pl.ds(0, _C1)], sems[slot]).wait()
        pltpu.make_async_copy(table_hbm.at[idx_v.at[pl.ds(0, _C2)]],
                              bufs[slot].at[pl.ds(_C1, _C2)], sems[slot]).wait()

    issue(0, 0)

    @pl.loop(0, _BPW, step=2)
    def _bag_pair(i):
        for b in range(2):
            ib = i + b
            buf = bufs[b]

            @pl.when(ib + 1 < _BPW)
            def _():
                issue(ib + 1, 1 - b)

            drain(b)

            def red(r, acc):
                a = list(acc)
                for u in range(8):
                    j = r * 8 + u
                    for k in range(4):
                        a[k] = a[k] + buf[j, pl.ds(k * 16, 16)]
                return tuple(a)

            acc = tuple(jnp.zeros((16,), jnp.float32) for _ in range(4))
            acc = lax.fori_loop(0, _L // 8, red, acc)
            for k in range(4):
                out_v[ib, pl.ds(k * 16, 16)] = acc[k]

    pltpu.sync_copy(out_v, out_hbm.at[pl.ds(base, _BPW)])


def _embbag_sums(text_flat, emb_weight):
    mesh = plsc.VectorSubcoreMesh(core_axis_name="c", subcore_axis_name="s",
                                  num_cores=_NC, num_subcores=_NS)
    f = pl.kernel(
        _embbag_body,
        out_type=jax.ShapeDtypeStruct((_B, _H), jnp.float32),
        mesh=mesh,
        name="embbag_sums",
        scratch_types=[
            pltpu.VMEM((_BPW * _L,), jnp.int32),
            pltpu.VMEM((_L, _H), jnp.float32),
            pltpu.VMEM((_L, _H), jnp.float32),
            pltpu.VMEM((_BPW, _H), jnp.float32),
            pltpu.SemaphoreType.DMA,
            pltpu.SemaphoreType.DMA,
        ],
        compiler_params=pltpu.CompilerParams(use_tc_tiling_on_sc=False),
    )
    return f(text_flat, emb_weight)


def _linear_body(x_ref, w_ref, b_ref, o_ref):
    o_ref[...] = (
        lax.dot_general(x_ref[...], w_ref[...], (((1,), (1,)), ((), ())),
                        preferred_element_type=jnp.float32) * (1.0 / _L)
        + b_ref[...]
    )


def kernel(text, emb_weight, fc1_weight, fc1_bias):
    sums = _embbag_sums(text.reshape(-1), emb_weight)
    out = pl.pallas_call(
        _linear_body,
        out_shape=jax.ShapeDtypeStruct((_B, _NOUT), jnp.float32),
    )(sums, fc1_weight, fc1_bias.reshape(1, _NOUT))
    return out
